# baseline (speedup 1.0000x reference)
import functools
import jax
import jax.numpy as jnp
import numpy as np
from jax.experimental import pallas as pl
from jax.experimental.pallas import tpu as pltpu

NPTS = 10000
NPAD = 10240
KNN = 16
NBLK = 7
CH = 64
IN_CH_PAIR = 6


def _dist_body(hb_ref, ht_ref, sqr_ref, sq_ref, cmask_ref, out_ref):
    hb = hb_ref[...]
    d = jnp.dot(hb, ht_ref[...], preferred_element_type=jnp.float32)
    sq_row = sqr_ref[...].reshape(-1, 1)
    out_ref[...] = sq_row + sq_ref[...] - 2.0 * d + cmask_ref[...]


def _pairwise_dists(h_pad):
    c = h_pad.shape[1]
    sq = jnp.sum(h_pad * h_pad, axis=1)
    cmask = jnp.where(jnp.arange(NPAD)[None, :] < NPTS, 0.0, jnp.inf).astype(jnp.float32)
    br = 128
    return pl.pallas_call(
        _dist_body,
        grid=(NPAD // br,),
        in_specs=[
            pl.BlockSpec((br, c), lambda i: (i, 0)),
            pl.BlockSpec((c, NPAD), lambda i: (0, 0)),
            pl.BlockSpec((1, 1, br), lambda i: (i, 0, 0)),
            pl.BlockSpec((1, NPAD), lambda i: (0, 0)),
            pl.BlockSpec((1, NPAD), lambda i: (0, 0)),
        ],
        out_specs=pl.BlockSpec((br, NPAD), lambda i: (i, 0)),
        out_shape=jax.ShapeDtypeStruct((NPAD, NPAD), jnp.float32),
    )(h_pad, h_pad.T, sq.reshape(NPAD // br, 1, br), sq[None, :], cmask)


def _seqmm(x, wt, br=512):
    cin, cout = wt.shape

    def body(x_ref, w_ref, o_ref):
        o_ref[...] = jnp.dot(x_ref[...], w_ref[...], preferred_element_type=jnp.float32)

    return pl.pallas_call(
        body,
        grid=(NPAD // br,),
        in_specs=[
            pl.BlockSpec((br, cin), lambda i: (i, 0)),
            pl.BlockSpec((cin, cout), lambda i: (0, 0)),
        ],
        out_specs=pl.BlockSpec((br, cout), lambda i: (i, 0)),
        out_shape=jax.ShapeDtypeStruct((NPAD, cout), jnp.float32),
    )(x, wt)


def _ecmax_body(feat_ref, w_ref, o_ref):
    w = w_ref[...]

    def kstep(k, m):
        t = jnp.dot(feat_ref[k], w, preferred_element_type=jnp.float32)
        return jnp.maximum(m, t)

    init = jnp.full(o_ref.shape, -jnp.inf, jnp.float32)
    m = jax.lax.fori_loop(0, KNN, kstep, init)
    o_ref[...] = jnp.maximum(m, 0.0)


def _edgeconv_max(feats_k, WT):
    cin = WT.shape[0]
    br = 256
    return pl.pallas_call(
        _ecmax_body,
        grid=(NPAD // br,),
        in_specs=[
            pl.BlockSpec((KNN, br, cin), lambda i: (0, i, 0)),
            pl.BlockSpec((cin, 64), lambda i: (0, 0)),
        ],
        out_specs=pl.BlockSpec((br, 64), lambda i: (i, 0)),
        out_shape=jax.ShapeDtypeStruct((NPAD, 64), jnp.float32),
    )(feats_k, WT)


def _fuse_body(f_ref, w_ref, b_ref, out_ref):
    h = jnp.dot(f_ref[...], w_ref[...], preferred_element_type=jnp.float32) + b_ref[...]
    h = jnp.maximum(h, 0.0)
    m = jnp.max(h, axis=1)
    out_ref[...] = m.reshape(out_ref.shape)


def _fuse_max(F_pad, W_fuse, b_fuse):
    fin = F_pad.shape[1]
    br = 512
    out = pl.pallas_call(
        _fuse_body,
        grid=(NPAD // br,),
        in_specs=[
            pl.BlockSpec((br, fin), lambda i: (i, 0)),
            pl.BlockSpec((fin, 1024), lambda i: (0, 0)),
            pl.BlockSpec((1, 1024), lambda i: (0, 0)),
        ],
        out_specs=pl.BlockSpec((1, br // 128, 128), lambda i: (i, 0, 0)),
        out_shape=jax.ShapeDtypeStruct((NPAD // br, br // 128, 128), jnp.float32),
    )(F_pad, W_fuse.T, b_fuse[None, :])
    return out.reshape(NPAD)


def _knn_idx(d, k, dilation):
    _, idx = jax.lax.top_k(-d, k * dilation)
    return idx[:, ::dilation]


def _spectral_norm(W, iters=3):
    u = jnp.ones((W.shape[0],), W.dtype) / jnp.sqrt(jnp.asarray(W.shape[0], W.dtype))
    for _ in range(iters):
        v = W.T @ u
        v = v / (jnp.linalg.norm(v) + 1e-12)
        u = W @ v
        u = u / (jnp.linalg.norm(u) + 1e-12)
    sigma = u @ (W @ v)
    return W / sigma


def _gather_feats(x, idx, cpad):
    x_j = x[idx]
    x_i = jnp.broadcast_to(x[:, None, :], x_j.shape)
    feat = jnp.concatenate([x_i, x_j - x_i], axis=-1)
    feat = jnp.transpose(feat, (1, 0, 2))
    return jnp.pad(feat, ((0, 0), (0, NPAD - NPTS), (0, cpad)))


def kernel(pos, x, W_head, b_head, W_blocks, b_blocks, W_fuse, b_fuse, W1, b1, W2, b2):
    padr = NPAD - NPTS
    data = jnp.concatenate([pos, x], axis=1)
    data_pad = jnp.pad(data, ((0, padr), (0, 2)))
    pos_pad = jnp.pad(pos, ((0, padr), (0, 5)))

    d0 = _pairwise_dists(pos_pad)[:NPTS]
    idx0 = _knn_idx(d0, KNN, 1)
    WhT = jnp.pad(W_head.T, ((0, 4), (0, 0)))
    fk = _gather_feats(data, idx0, 4)
    feat = _edgeconv_max(fk, WhT)[:NPTS]
    feats = [feat]

    for i in range(NBLK - 1):
        h = feats[-1]
        h_pad = jnp.pad(h, ((0, padr), (0, 0)))
        d = _pairwise_dists(h_pad)[:NPTS]
        idx = _knn_idx(d, KNN, 1 + i)
        fk = _gather_feats(h, idx, 0)
        feats.append(_edgeconv_max(fk, W_blocks[i].T)[:NPTS] + h)

    F = jnp.concatenate(feats, axis=1)
    F_pad = jnp.pad(F, ((0, padr), (0, 0)))
    vmax = _fuse_max(F_pad, W_fuse, b_fuse)[:NPTS]

    h1 = _spectral_norm(W1) @ vmax + b1
    out = _spectral_norm(W2) @ h1 + b2
    return out[None, :]


# speedup vs baseline: 1.0046x; 1.0046x over previous
import functools
import jax
import jax.numpy as jnp
import numpy as np
from jax.experimental import pallas as pl
from jax.experimental.pallas import tpu as pltpu

NPTS = 10000
NPAD = 10240
KNN = 16
NBLK = 7
CH = 64
IN_CH_PAIR = 6


def _dist_body(hb_ref, ht_ref, sqr_ref, sq_ref, cmask_ref, out_ref):
    hb = hb_ref[...]
    d = jnp.dot(hb, ht_ref[...], preferred_element_type=jnp.float32)
    sq_row = sqr_ref[...].reshape(-1, 1)
    out_ref[...] = sq_row + sq_ref[...] - 2.0 * d + cmask_ref[...]


def _pairwise_dists(h_pad):
    c = h_pad.shape[1]
    sq = jnp.sum(h_pad * h_pad, axis=1)
    cmask = jnp.where(jnp.arange(NPAD)[None, :] < NPTS, 0.0, jnp.inf).astype(jnp.float32)
    br = 128
    return pl.pallas_call(
        _dist_body,
        grid=(NPAD // br,),
        in_specs=[
            pl.BlockSpec((br, c), lambda i: (i, 0)),
            pl.BlockSpec((c, NPAD), lambda i: (0, 0)),
            pl.BlockSpec((1, 1, br), lambda i: (i, 0, 0)),
            pl.BlockSpec((1, NPAD), lambda i: (0, 0)),
            pl.BlockSpec((1, NPAD), lambda i: (0, 0)),
        ],
        out_specs=pl.BlockSpec((br, NPAD), lambda i: (i, 0)),
        out_shape=jax.ShapeDtypeStruct((NPAD, NPAD), jnp.float32),
    )(h_pad, h_pad.T, sq.reshape(NPAD // br, 1, br), sq[None, :], cmask)


def _seqmm(x, wt, br=512):
    cin, cout = wt.shape

    def body(x_ref, w_ref, o_ref):
        o_ref[...] = jnp.dot(x_ref[...], w_ref[...], preferred_element_type=jnp.float32)

    return pl.pallas_call(
        body,
        grid=(NPAD // br,),
        in_specs=[
            pl.BlockSpec((br, cin), lambda i: (i, 0)),
            pl.BlockSpec((cin, cout), lambda i: (0, 0)),
        ],
        out_specs=pl.BlockSpec((br, cout), lambda i: (i, 0)),
        out_shape=jax.ShapeDtypeStruct((NPAD, cout), jnp.float32),
    )(x, wt)


def _ecmax_body(feat_ref, w_ref, o_ref):
    w = w_ref[...]

    def kstep(k, m):
        t = jnp.dot(feat_ref[k], w, preferred_element_type=jnp.float32)
        return jnp.maximum(m, t)

    init = jnp.full(o_ref.shape, -jnp.inf, jnp.float32)
    m = jax.lax.fori_loop(0, KNN, kstep, init)
    o_ref[...] = jnp.maximum(m, 0.0)


def _edgeconv_max(feats_k, WT):
    cin = WT.shape[0]
    br = 256
    return pl.pallas_call(
        _ecmax_body,
        grid=(NPAD // br,),
        in_specs=[
            pl.BlockSpec((KNN, br, cin), lambda i: (0, i, 0)),
            pl.BlockSpec((cin, 64), lambda i: (0, 0)),
        ],
        out_specs=pl.BlockSpec((br, 64), lambda i: (i, 0)),
        out_shape=jax.ShapeDtypeStruct((NPAD, 64), jnp.float32),
    )(feats_k, WT)


def _fuse_body(f_ref, w_ref, b_ref, out_ref):
    h = jnp.dot(f_ref[...], w_ref[...], preferred_element_type=jnp.float32) + b_ref[...]
    h = jnp.maximum(h, 0.0)
    m = jnp.max(h, axis=1)
    out_ref[...] = m.reshape(out_ref.shape)


def _fuse_max(F_pad, W_fuse, b_fuse):
    fin = F_pad.shape[1]
    br = 512
    out = pl.pallas_call(
        _fuse_body,
        grid=(NPAD // br,),
        in_specs=[
            pl.BlockSpec((br, fin), lambda i: (i, 0)),
            pl.BlockSpec((fin, 1024), lambda i: (0, 0)),
            pl.BlockSpec((1, 1024), lambda i: (0, 0)),
        ],
        out_specs=pl.BlockSpec((1, br // 128, 128), lambda i: (i, 0, 0)),
        out_shape=jax.ShapeDtypeStruct((NPAD // br, br // 128, 128), jnp.float32),
    )(F_pad, W_fuse.T, b_fuse[None, :])
    return out.reshape(NPAD)


def _knn_idx(d, k, dilation):
    _, idx = jax.lax.top_k(-d, k * dilation)
    return idx[:, ::dilation]


def _spectral_norm(W, iters=3):
    u = jnp.ones((W.shape[0],), W.dtype) / jnp.sqrt(jnp.asarray(W.shape[0], W.dtype))
    for _ in range(iters):
        v = W.T @ u
        v = v / (jnp.linalg.norm(v) + 1e-12)
        u = W @ v
        u = u / (jnp.linalg.norm(u) + 1e-12)
    sigma = u @ (W @ v)
    return W / sigma


def _gather_feats(x, idx, cpad):
    x_j = x[idx]
    x_i = jnp.broadcast_to(x[:, None, :], x_j.shape)
    feat = jnp.concatenate([x_i, x_j - x_i], axis=-1)
    feat = jnp.transpose(feat, (1, 0, 2))
    return jnp.pad(feat, ((0, 0), (0, NPAD - NPTS), (0, cpad)))


def kernel(pos, x, W_head, b_head, W_blocks, b_blocks, W_fuse, b_fuse, W1, b1, W2, b2):
    padr = NPAD - NPTS
    data = jnp.concatenate([pos, x], axis=1)
    data_pad = jnp.pad(data, ((0, padr), (0, 2)))
    pos_pad = jnp.pad(pos, ((0, padr), (0, 5)))

    d0 = _pairwise_dists(pos_pad)[:NPTS, :NPTS]
    idx0 = _knn_idx(d0, KNN, 1)
    WhT = jnp.pad(W_head.T, ((0, 4), (0, 0)))
    fk = _gather_feats(data, idx0, 4)
    feat = _edgeconv_max(fk, WhT)[:NPTS]
    feats = [feat]

    for i in range(NBLK - 1):
        h = feats[-1]
        h_pad = jnp.pad(h, ((0, padr), (0, 0)))
        d = _pairwise_dists(h_pad)[:NPTS, :NPTS]
        idx = _knn_idx(d, KNN, 1 + i)
        fk = _gather_feats(h, idx, 0)
        feats.append(_edgeconv_max(fk, W_blocks[i].T)[:NPTS] + h)

    F = jnp.concatenate(feats, axis=1)
    F_pad = jnp.pad(F, ((0, padr), (0, 0)))
    vmax = _fuse_max(F_pad, W_fuse, b_fuse)[:NPTS]

    h1 = _spectral_norm(W1) @ vmax + b1
    out = _spectral_norm(W2) @ h1 + b2
    return out[None, :]


# speedup vs baseline: 1.0050x; 1.0004x over previous
import functools
import jax
import jax.numpy as jnp
import numpy as np
from jax.experimental import pallas as pl
from jax.experimental.pallas import tpu as pltpu

NPTS = 10000
NPAD = 10240
KNN = 16
NBLK = 7
CH = 64
IN_CH_PAIR = 6


def _dist_body(hb_ref, ht_ref, sqr_ref, sq_ref, out_ref):
    hb = hb_ref[...]
    d = jnp.dot(hb, ht_ref[...], preferred_element_type=jnp.float32)
    sq_row = sqr_ref[...].reshape(-1, 1)
    out_ref[...] = -(sq_row + sq_ref[...] - 2.0 * d)


def _pairwise_dists(h_pad):
    c = h_pad.shape[1]
    sq = jnp.sum(h_pad * h_pad, axis=1)
    br = 128
    return pl.pallas_call(
        _dist_body,
        grid=(NPAD // br,),
        in_specs=[
            pl.BlockSpec((br, c), lambda i: (i, 0)),
            pl.BlockSpec((c, NPAD), lambda i: (0, 0)),
            pl.BlockSpec((1, 1, br), lambda i: (i, 0, 0)),
            pl.BlockSpec((1, NPAD), lambda i: (0, 0)),
        ],
        out_specs=pl.BlockSpec((br, NPAD), lambda i: (i, 0)),
        out_shape=jax.ShapeDtypeStruct((NPAD, NPAD), jnp.float32),
    )(h_pad, h_pad.T, sq.reshape(NPAD // br, 1, br), sq[None, :])


def _seqmm(x, wt, br=512):
    cin, cout = wt.shape

    def body(x_ref, w_ref, o_ref):
        o_ref[...] = jnp.dot(x_ref[...], w_ref[...], preferred_element_type=jnp.float32)

    return pl.pallas_call(
        body,
        grid=(NPAD // br,),
        in_specs=[
            pl.BlockSpec((br, cin), lambda i: (i, 0)),
            pl.BlockSpec((cin, cout), lambda i: (0, 0)),
        ],
        out_specs=pl.BlockSpec((br, cout), lambda i: (i, 0)),
        out_shape=jax.ShapeDtypeStruct((NPAD, cout), jnp.float32),
    )(x, wt)


def _ecmax_body(feat_ref, w_ref, o_ref):
    w = w_ref[...]

    def kstep(k, m):
        t = jnp.dot(feat_ref[k], w, preferred_element_type=jnp.float32)
        return jnp.maximum(m, t)

    init = jnp.full(o_ref.shape, -jnp.inf, jnp.float32)
    m = jax.lax.fori_loop(0, KNN, kstep, init)
    o_ref[...] = jnp.maximum(m, 0.0)


def _edgeconv_max(feats_k, WT):
    cin = WT.shape[0]
    br = 256
    return pl.pallas_call(
        _ecmax_body,
        grid=(NPAD // br,),
        in_specs=[
            pl.BlockSpec((KNN, br, cin), lambda i: (0, i, 0)),
            pl.BlockSpec((cin, 64), lambda i: (0, 0)),
        ],
        out_specs=pl.BlockSpec((br, 64), lambda i: (i, 0)),
        out_shape=jax.ShapeDtypeStruct((NPAD, 64), jnp.float32),
    )(feats_k, WT)


def _fuse_body(f_ref, w_ref, b_ref, out_ref):
    h = jnp.dot(f_ref[...], w_ref[...], preferred_element_type=jnp.float32) + b_ref[...]
    h = jnp.maximum(h, 0.0)
    m = jnp.max(h, axis=1)
    out_ref[...] = m.reshape(out_ref.shape)


def _fuse_max(F_pad, W_fuse, b_fuse):
    fin = F_pad.shape[1]
    br = 512
    out = pl.pallas_call(
        _fuse_body,
        grid=(NPAD // br,),
        in_specs=[
            pl.BlockSpec((br, fin), lambda i: (i, 0)),
            pl.BlockSpec((fin, 1024), lambda i: (0, 0)),
            pl.BlockSpec((1, 1024), lambda i: (0, 0)),
        ],
        out_specs=pl.BlockSpec((1, br // 128, 128), lambda i: (i, 0, 0)),
        out_shape=jax.ShapeDtypeStruct((NPAD // br, br // 128, 128), jnp.float32),
    )(F_pad, W_fuse.T, b_fuse[None, :])
    return out.reshape(NPAD)


def _knn_idx(dneg, k, dilation):
    _, idx = jax.lax.top_k(dneg, k * dilation)
    return idx[:, ::dilation]


def _spectral_norm(W, iters=3):
    u = jnp.ones((W.shape[0],), W.dtype) / jnp.sqrt(jnp.asarray(W.shape[0], W.dtype))
    for _ in range(iters):
        v = W.T @ u
        v = v / (jnp.linalg.norm(v) + 1e-12)
        u = W @ v
        u = u / (jnp.linalg.norm(u) + 1e-12)
    sigma = u @ (W @ v)
    return W / sigma


def _gather_feats(x, idx, cpad):
    x_j = x[idx]
    x_i = jnp.broadcast_to(x[:, None, :], x_j.shape)
    feat = jnp.concatenate([x_i, x_j - x_i], axis=-1)
    feat = jnp.transpose(feat, (1, 0, 2))
    return jnp.pad(feat, ((0, 0), (0, NPAD - NPTS), (0, cpad)))


def kernel(pos, x, W_head, b_head, W_blocks, b_blocks, W_fuse, b_fuse, W1, b1, W2, b2):
    padr = NPAD - NPTS
    data = jnp.concatenate([pos, x], axis=1)
    data_pad = jnp.pad(data, ((0, padr), (0, 2)))
    pos_pad = jnp.pad(pos, ((0, padr), (0, 5)))

    d0 = _pairwise_dists(pos_pad)[:NPTS, :NPTS]
    idx0 = _knn_idx(d0, KNN, 1)
    WhT = jnp.pad(W_head.T, ((0, 4), (0, 0)))
    fk = _gather_feats(data, idx0, 4)
    feat = _edgeconv_max(fk, WhT)[:NPTS]
    feats = [feat]

    for i in range(NBLK - 1):
        h = feats[-1]
        h_pad = jnp.pad(h, ((0, padr), (0, 0)))
        d = _pairwise_dists(h_pad)[:NPTS, :NPTS]
        idx = _knn_idx(d, KNN, 1 + i)
        fk = _gather_feats(h, idx, 0)
        feats.append(_edgeconv_max(fk, W_blocks[i].T)[:NPTS] + h)

    F = jnp.concatenate(feats, axis=1)
    F_pad = jnp.pad(F, ((0, padr), (0, 0)))
    vmax = _fuse_max(F_pad, W_fuse, b_fuse)[:NPTS]

    h1 = _spectral_norm(W1) @ vmax + b1
    out = _spectral_norm(W2) @ h1 + b2
    return out[None, :]
